# Initial kernel scaffold; baseline (speedup 1.0000x reference)
import jax
import jax.numpy as jnp
from jax.experimental import pallas as pl


def kernel(x, edge_index, W1, a_src1, a_dst1, b1, W2, a_src2, a_dst2, b2, W3, a_src3, a_dst3, b3):
    raise NotImplementedError("write your pallas kernel here")


# speedup vs baseline: 1.0002x; 1.0002x over previous
import jax
import jax.numpy as jnp
from jax.experimental import pallas as pl

N = 10000
E = 320000
IN = 12
H = 8
C = 64
OUT = 5


def _gat_conv(x, src, dst, W, a_s, a_d, b, heads, ch, concat):
    n = x.shape[0]
    h = (x @ W).reshape(n, heads, ch)
    alpha_s = jnp.sum(h * a_s[None, :, :], axis=-1)
    alpha_d = jnp.sum(h * a_d[None, :, :], axis=-1)
    e = alpha_s[src] + alpha_d[dst]
    e = jnp.where(e > 0, e, 0.2 * e)
    emax = jax.ops.segment_max(e, dst, num_segments=n)
    emax = jnp.where(jnp.isfinite(emax), emax, 0.0)
    p = jnp.exp(e - emax[dst])
    s = jax.ops.segment_sum(p, dst, num_segments=n)
    alpha = p / (s[dst] + 1e-16)
    msg = h[src] * alpha[:, :, None]
    out = jax.ops.segment_sum(msg, dst, num_segments=n)
    if concat:
        out = out.reshape(n, heads * ch)
    else:
        out = out.mean(axis=1)
    return out + b


def _logsoftmax_body(x_ref, o_ref):
    x = x_ref[...]
    m = jnp.max(x, axis=1, keepdims=True)
    lse = jnp.log(jnp.sum(jnp.exp(x - m), axis=1, keepdims=True)) + m
    o_ref[...] = x - lse


def kernel(x, edge_index, W1, a_src1, a_dst1, b1, W2, a_src2, a_dst2, b2, W3, a_src3, a_dst3, b3):
    loops = jnp.arange(N, dtype=edge_index.dtype)
    src = jnp.concatenate([edge_index[0], loops])
    dst = jnp.concatenate([edge_index[1], loops])
    h = _gat_conv(x, src, dst, W1, a_src1, a_dst1, b1, H, C, True)
    h = jax.nn.relu(h)
    h = _gat_conv(h, src, dst, W2, a_src2, a_dst2, b2, H, C, True)
    h = jax.nn.relu(h)
    h = _gat_conv(h, src, dst, W3, a_src3, a_dst3, b3, H, OUT, False)
    return pl.pallas_call(
        _logsoftmax_body,
        out_shape=jax.ShapeDtypeStruct((N, OUT), jnp.float32),
    )(h)


# speedup vs baseline: 6.8048x; 6.8035x over previous
import functools

import jax
import jax.numpy as jnp
from jax import lax
from jax.experimental import pallas as pl
from jax.experimental.pallas import tpu as pltpu
from jax.experimental.pallas import tpu_sc as plsc

N = 10000
E = 320000
IN = 12
H = 8
C = 64
OUT = 5

NPAD = 10240
E2 = E + N
NS = 16
E2P = 330752
E2PX = E2P + 64
BN = 512
ROWS_T = NPAD // NS
F32 = jnp.float32
I32 = jnp.int32


def _mm12_body(kb, apply_relu, x_ref, w_ref, b_ref, h4_ref, al_ref):
    acc = jnp.zeros((BN, 640), F32)
    for k in range(kb):
        xk = x_ref[k]
        if apply_relu:
            xk = jnp.maximum(xk + b_ref[k][None, :], 0.0)
        acc = acc + jnp.dot(xk, w_ref[k], preferred_element_type=F32)
    for j in range(4):
        h4_ref[j] = acc[:, 128 * j:128 * (j + 1)]
    al_ref[...] = acc[:, 512:640]


def _tc_mm12(x4, wk, bk, apply_relu):
    kb = x4.shape[0]
    return pl.pallas_call(
        functools.partial(_mm12_body, kb, apply_relu),
        grid=(NPAD // BN,),
        in_specs=[pl.BlockSpec((kb, BN, 128), lambda i: (0, i, 0)),
                  pl.BlockSpec((kb, 128, 640), lambda i: (0, 0, 0)),
                  pl.BlockSpec((kb, 128), lambda i: (0, 0))],
        out_specs=[pl.BlockSpec((4, BN, 128), lambda i: (0, i, 0)),
                   pl.BlockSpec((BN, 128), lambda i: (i, 0))],
        out_shape=[jax.ShapeDtypeStruct((4, NPAD, 128), F32),
                   jax.ShapeDtypeStruct((NPAD, 128), F32)],
    )(x4, wk, bk)


def _mm3_body(x_ref, w_ref, b_ref, h3_ref, al_ref):
    acc = jnp.zeros((BN, 128), F32)
    for k in range(4):
        xk = jnp.maximum(x_ref[k] + b_ref[k][None, :], 0.0)
        acc = acc + jnp.dot(xk, w_ref[k], preferred_element_type=F32)
    z11 = jnp.zeros((BN, 11), F32)
    parts = []
    for h in range(8):
        parts.append(acc[:, 5 * h:5 * h + 5])
        parts.append(z11)
    h3_ref[...] = jnp.concatenate(parts, axis=1)
    al_ref[...] = acc


def _tc_mm3(x4, wk, bk):
    return pl.pallas_call(
        _mm3_body,
        grid=(NPAD // BN,),
        in_specs=[pl.BlockSpec((4, BN, 128), lambda i: (0, i, 0)),
                  pl.BlockSpec((4, 128, 128), lambda i: (0, 0, 0)),
                  pl.BlockSpec((4, 128), lambda i: (0, 0))],
        out_specs=[pl.BlockSpec((BN, 128), lambda i: (i, 0)),
                   pl.BlockSpec((BN, 128), lambda i: (i, 0))],
        out_shape=[jax.ShapeDtypeStruct((NPAD, 128), F32),
                   jax.ShapeDtypeStruct((NPAD, 128), F32)],
    )(x4, wk, bk)


def _tr_body(x_ref, o_ref):
    o_ref[...] = x_ref[...].T


def _tc_transpose(al):
    return pl.pallas_call(
        _tr_body,
        grid=(NPAD // BN,),
        in_specs=[pl.BlockSpec((BN, 128), lambda i: (i, 0))],
        out_specs=pl.BlockSpec((128, BN), lambda i: (0, i)),
        out_shape=jax.ShapeDtypeStruct((128, NPAD), F32),
    )(al)


def _fin_body(x_ref, b_ref, o_ref):
    sc2 = x_ref[0] + x_ref[1]
    hsum = (sc2[:, 0:8] + sc2[:, 16:24] + sc2[:, 32:40] + sc2[:, 48:56])
    z = hsum * 0.125 + b_ref[0:1, :]
    msk = lax.broadcasted_iota(I32, (1, 8), 1) < OUT
    zm = jnp.where(msk, z, -1e30)
    m = jnp.max(zm, axis=1, keepdims=True)
    lse = jnp.log(jnp.sum(jnp.where(msk, jnp.exp(z - m), 0.0),
                          axis=1, keepdims=True)) + m
    o_ref[...] = z - lse


def _tc_final(x4, b3p):
    return pl.pallas_call(
        _fin_body,
        grid=(NPAD // BN,),
        in_specs=[pl.BlockSpec((2, BN, 64), lambda i: (0, i, 0)),
                  pl.BlockSpec((8, 8), lambda i: (0, 0))],
        out_specs=pl.BlockSpec((BN, 8), lambda i: (i, 0)),
        out_shape=jax.ShapeDtypeStruct((NPAD, 8), F32),
    )(x4, b3p)


def _lane0(v):
    return jnp.sum(jnp.where(lax.iota(I32, 16) == 0, v, 0))


def _sc_edge_kernel(a_s_off, a_d_off):
    D = 128
    mesh = plsc.VectorSubcoreMesh(core_axis_name="c", subcore_axis_name="s")

    @functools.partial(
        pl.kernel,
        mesh=mesh,
        compiler_params=pltpu.CompilerParams(needs_layout_passes=False),
        out_type=jax.ShapeDtypeStruct((4 * NPAD, D), F32),
        scratch_types=[
            pltpu.VMEM((16,), I32),
            pltpu.VMEM((16,), I32),
            pltpu.VMEM((64,), I32),
            pltpu.VMEM((64,), I32),
            pltpu.VMEM((64,), I32),
            pltpu.VMEM((ROWS_T * 16,), F32),
            pltpu.VMEM((32,), F32),
            pltpu.VMEM((64, D), F32),
            pltpu.VMEM((NPAD,), F32),
            pltpu.VMEM((NPAD,), F32),
            pltpu.VMEM((ROWS_T,), F32),
            pltpu.VMEM((ROWS_T,), F32),
            pltpu.VMEM((ROWS_T, D), F32),
            pltpu.SemaphoreType.DMA,
        ],
    )
    def k(h4, alT, srcs, dsts, rs, out4,
          rsb0, rsb1, srcb, srcb2, dstb, s16, abuf, rowb,
          aS0, aS1, aDl0, aDl1, acc, sem):
        cid = lax.axis_index("c")
        tid = lax.axis_index("s")
        base = tid * ROWS_T
        iot = lax.iota(I32, 16)
        zv = jnp.zeros((16,), F32)

        pltpu.sync_copy(rs.at[pl.ds(base, 16)], rsb0)
        pltpu.sync_copy(rs.at[pl.ds(base + ROWS_T, 16)], rsb1)
        e0 = rsb0[...][0]
        e1 = rsb1[...][0]
        a0 = (e0 // 64) * 64
        nch = (e1 - a0 + 63) // 64

        def zero_s(i, _):
            s16[pl.ds(i * 16, 16)] = zv
            return 0
        lax.fori_loop(0, ROWS_T, zero_s, 0)

        for r in range(2):
            q = cid * 2 + r

            if True:
                pltpu.sync_copy(alT.at[pl.ds((a_s_off + 2 * q) * NPAD, NPAD)], aS0)
                pltpu.sync_copy(alT.at[pl.ds((a_s_off + 2 * q + 1) * NPAD, NPAD)], aS1)
                pltpu.sync_copy(
                    alT.at[pl.ds((a_d_off + 2 * q) * NPAD + base, ROWS_T)], aDl0)
                pltpu.sync_copy(
                    alT.at[pl.ds((a_d_off + 2 * q + 1) * NPAD + base, ROWS_T)], aDl1)
                oh0 = jnp.where(iot == 2 * r, 1.0, 0.0)
                oh1 = jnp.where(iot == 2 * r + 1, 1.0, 0.0)

                def chunk_a(ci, _):
                    off = a0 + ci * 64
                    pltpu.sync_copy(srcs.at[pl.ds(off, 64)], srcb)
                    pltpu.sync_copy(dsts.at[pl.ds(off, 64)], dstb)
                    for g in range(4):
                        ge = off + g * 16 + iot
                        ownf = jnp.where(
                            jnp.logical_and(ge >= e0, ge < e1), 1.0, 0.0)
                        sv = srcb[pl.ds(g * 16, 16)]
                        dv = dstb[pl.ds(g * 16, 16)]
                        dlv = jnp.clip(dv - base, 0, ROWS_T - 1)
                        e_0 = plsc.load_gather(aS0, [sv]) + plsc.load_gather(aDl0, [dlv])
                        e_0 = jnp.where(e_0 > 0, e_0, 0.2 * e_0)
                        pm0 = jnp.exp(e_0) * ownf
                        e_1 = plsc.load_gather(aS1, [sv]) + plsc.load_gather(aDl1, [dlv])
                        e_1 = jnp.where(e_1 > 0, e_1, 0.2 * e_1)
                        pm1 = jnp.exp(e_1) * ownf
                        slv = dlv * 16
                        for i in range(16):
                            sl = slv[i]
                            s16[pl.ds(sl, 16)] = (s16[pl.ds(sl, 16)]
                                                  + pm0[i] * oh0
                                                  + pm1[i] * oh1)
                    return 0

                lax.fori_loop(0, nch, chunk_a, 0)

            if True:
                def zero_acc(i, _):
                    for v in range(D // 16):
                        acc[i, pl.ds(v * 16, 16)] = zv
                    return 0
                lax.fori_loop(0, ROWS_T, zero_acc, 0)

                def chunk_b(ci, _):
                    off = a0 + ci * 64
                    pltpu.sync_copy(srcs.at[pl.ds(off, 64)], srcb)
                    pltpu.sync_copy(dsts.at[pl.ds(off, 64)], dstb)
                    for g in range(4):
                        srcb2[pl.ds(g * 16, 16)] = srcb[pl.ds(g * 16, 16)] + q * NPAD
                    cp = pltpu.async_copy(h4.at[srcb2], rowb, sem)
                    cp.wait()
                    for g in range(4):
                        ge = off + g * 16 + iot
                        ownv = jnp.logical_and(ge >= e0, ge < e1)
                        sv = srcb[pl.ds(g * 16, 16)]
                        dv = dstb[pl.ds(g * 16, 16)]
                        dlv = jnp.clip(dv - base, 0, ROWS_T - 1)
                        e_0 = plsc.load_gather(aS0, [sv]) + plsc.load_gather(aDl0, [dlv])
                        e_0 = jnp.where(e_0 > 0, e_0, 0.2 * e_0)
                        p0 = jnp.exp(e_0)
                        e_1 = plsc.load_gather(aS1, [sv]) + plsc.load_gather(aDl1, [dlv])
                        e_1 = jnp.where(e_1 > 0, e_1, 0.2 * e_1)
                        p1 = jnp.exp(e_1)
                        s0 = plsc.load_gather(s16, [dlv * 16 + 2 * r])
                        s1 = plsc.load_gather(s16, [dlv * 16 + 2 * r + 1])
                        a_0 = jnp.where(ownv, p0 / (s0 + 1e-16), 0.0)
                        a_1 = jnp.where(ownv, p1 / (s1 + 1e-16), 0.0)
                        abuf[pl.ds(0, 16)] = a_0
                        abuf[pl.ds(16, 16)] = a_1
                        for l in range(16):
                            ei = g * 16 + l
                            br0 = plsc.load_gather(abuf, [jnp.full((16,), l, I32)])
                            br1 = plsc.load_gather(abuf, [jnp.full((16,), 16 + l, I32)])
                            dl = dlv[l]
                            for v in range(8):
                                sc = br0 if v < 4 else br1
                                acc[dl, pl.ds(v * 16, 16)] = (
                                    acc[dl, pl.ds(v * 16, 16)]
                                    + rowb[ei, pl.ds(v * 16, 16)] * sc)
                    return 0

                lax.fori_loop(0, nch, chunk_b, 0)
                pltpu.sync_copy(acc,
                                out4.at[pl.ds(q * NPAD + base, ROWS_T)])

    return k


_sc_layer12 = _sc_edge_kernel(0, 8)

def _sc_edge3_kernel():
    mesh = plsc.VectorSubcoreMesh(core_axis_name="c", subcore_axis_name="s")

    @functools.partial(
        pl.kernel,
        mesh=mesh,
        compiler_params=pltpu.CompilerParams(needs_layout_passes=False),
        out_type=jax.ShapeDtypeStruct((2 * NPAD * 64,), F32),
        scratch_types=[
            pltpu.VMEM((16,), I32),
            pltpu.VMEM((16,), I32),
            pltpu.VMEM((64,), I32),
            pltpu.VMEM((64,), I32),
            pltpu.VMEM((ROWS_T * 16,), F32),
            pltpu.VMEM((64,), F32),
            pltpu.VMEM((64, 128), F32),
            [pltpu.VMEM((NPAD,), F32)] * 4,
            [pltpu.VMEM((ROWS_T,), F32)] * 4,
            pltpu.VMEM((ROWS_T * 64,), F32),
            pltpu.SemaphoreType.DMA,
        ],
    )
    def k(h3w, alT, srcs, dsts, rs, out3,
          rsb0, rsb1, srcb, dstb, s16, abuf, rowb, aS, aDl, acc, sem):
        cid = lax.axis_index("c")
        tid = lax.axis_index("s")
        base = tid * ROWS_T
        iot = lax.iota(I32, 16)
        zv = jnp.zeros((16,), F32)

        pltpu.sync_copy(rs.at[pl.ds(base, 16)], rsb0)
        pltpu.sync_copy(rs.at[pl.ds(base + ROWS_T, 16)], rsb1)
        e0 = rsb0[...][0]
        e1 = rsb1[...][0]
        a0 = (e0 // 64) * 64
        nch = (e1 - a0 + 63) // 64

        def zero_s(i, _):
            s16[pl.ds(i * 16, 16)] = zv
            return 0
        lax.fori_loop(0, ROWS_T, zero_s, 0)

        for lh in range(4):
            pltpu.sync_copy(
                alT.at[pl.ds((40 + 4 * cid + lh) * NPAD, NPAD)], aS[lh])
            pltpu.sync_copy(
                alT.at[pl.ds((48 + 4 * cid + lh) * NPAD + base, ROWS_T)],
                aDl[lh])

        def edge_p(sv, dlv, lh):
            ee = (plsc.load_gather(aS[lh], [sv])
                  + plsc.load_gather(aDl[lh], [dlv]))
            ee = jnp.where(ee > 0, ee, 0.2 * ee)
            return jnp.exp(ee)

        ohs = [jnp.where(iot == lh, 1.0, 0.0) for lh in range(4)]

        def chunk_a(ci, _):
            off = a0 + ci * 64
            pltpu.sync_copy(srcs.at[pl.ds(off, 64)], srcb)
            pltpu.sync_copy(dsts.at[pl.ds(off, 64)], dstb)
            for g in range(4):
                ge = off + g * 16 + iot
                ownf = jnp.where(jnp.logical_and(ge >= e0, ge < e1), 1.0, 0.0)
                sv = srcb[pl.ds(g * 16, 16)]
                dv = dstb[pl.ds(g * 16, 16)]
                dlv = jnp.clip(dv - base, 0, ROWS_T - 1)
                pm = [edge_p(sv, dlv, lh) * ownf for lh in range(4)]
                slv = dlv * 16
                for i in range(16):
                    sl = slv[i]
                    s16[pl.ds(sl, 16)] = (
                        s16[pl.ds(sl, 16)] + pm[0][i] * ohs[0]
                        + pm[1][i] * ohs[1] + pm[2][i] * ohs[2]
                        + pm[3][i] * ohs[3])
            return 0

        lax.fori_loop(0, nch, chunk_a, 0)

        def zero_acc(i, _):
            for v in range(4):
                acc[pl.ds(i * 64 + v * 16, 16)] = zv
            return 0
        lax.fori_loop(0, ROWS_T, zero_acc, 0)

        def chunk_b(ci, _):
            off = a0 + ci * 64
            pltpu.sync_copy(srcs.at[pl.ds(off, 64)], srcb)
            pltpu.sync_copy(dsts.at[pl.ds(off, 64)], dstb)
            cp = pltpu.async_copy(h3w.at[srcb], rowb, sem)
            cp.wait()
            for g in range(4):
                ge = off + g * 16 + iot
                ownv = jnp.logical_and(ge >= e0, ge < e1)
                sv = srcb[pl.ds(g * 16, 16)]
                dv = dstb[pl.ds(g * 16, 16)]
                dlv = jnp.clip(dv - base, 0, ROWS_T - 1)
                for lh in range(4):
                    pv = edge_p(sv, dlv, lh)
                    sg = plsc.load_gather(s16, [dlv * 16 + lh])
                    av = jnp.where(ownv, pv / (sg + 1e-16), 0.0)
                    abuf[pl.ds(16 * lh, 16)] = av
                for l in range(16):
                    ei = g * 16 + l
                    dl = dlv[l]
                    for lh in range(4):
                        br = plsc.load_gather(
                            abuf, [jnp.full((16,), 16 * lh + l, I32)])
                        col = 64 * cid + 16 * lh
                        acc[pl.ds(dl * 64 + 16 * lh, 16)] = (
                            acc[pl.ds(dl * 64 + 16 * lh, 16)]
                            + rowb[ei, pl.ds(col, 16)] * br)
            return 0

        lax.fori_loop(0, nch, chunk_b, 0)
        pltpu.sync_copy(acc, out3.at[pl.ds((cid * NPAD + base) * 64,
                                           ROWS_T * 64)])

    return k


_sc_layer3 = _sc_edge3_kernel()


def _aug_w(W, a_s, a_d, ch):
    K = W.shape[0]
    Wr = W.reshape(K, H, ch)
    ws = jnp.einsum("khc,hc->kh", Wr, a_s)
    wd = jnp.einsum("khc,hc->kh", Wr, a_d)
    return ws, wd


def kernel(x, edge_index, W1, a_src1, a_dst1, b1,
           W2, a_src2, a_dst2, b2, W3, a_src3, a_dst3, b3):
    loops = jnp.arange(N, dtype=jnp.int32)
    padi = jnp.full((E2P - E2,), N, jnp.int32)
    src0 = jnp.concatenate([edge_index[0].astype(jnp.int32), loops, padi])
    dst0 = jnp.concatenate([edge_index[1].astype(jnp.int32), loops, padi])
    order = jnp.argsort(dst0)
    dsts = jnp.concatenate([dst0[order], jnp.full((64,), NPAD - 1, jnp.int32)])
    srcs = jnp.concatenate([src0[order], jnp.full((64,), N, jnp.int32)])
    rs = jnp.searchsorted(dsts[:E2P], jnp.arange(NPAD + 1, dtype=jnp.int32)
                          ).astype(jnp.int32)
    rs = jnp.concatenate([rs, jnp.full((15,), E2P, jnp.int32)])

    xp = jnp.zeros((1, NPAD, 128), F32).at[0, :N, :IN].set(x)
    ws1, wd1 = _aug_w(W1, a_src1, a_dst1, C)
    W1p = jnp.zeros((1, 128, 640), F32)
    W1p = W1p.at[0, :IN, :512].set(W1).at[0, :IN, 512:520].set(ws1)
    W1p = W1p.at[0, :IN, 520:528].set(wd1)
    b0 = jnp.zeros((1, 128), F32)
    h4_1, al1 = _tc_mm12(xp, W1p, b0, False)
    alT1 = _tc_transpose(al1).reshape(-1)
    out1 = _sc_layer12(h4_1.reshape(4 * NPAD, 128), alT1, srcs, dsts, rs)

    ws2, wd2 = _aug_w(W2, a_src2, a_dst2, C)
    W2p = jnp.concatenate(
        [W2.reshape(4, 128, 512), ws2.reshape(4, 128, 8),
         wd2.reshape(4, 128, 8), jnp.zeros((4, 128, 112), F32)], axis=2)
    h4_2, al2 = _tc_mm12(out1.reshape(4, NPAD, 128), W2p,
                         b1.reshape(4, 128), True)
    alT2 = _tc_transpose(al2).reshape(-1)
    out2 = _sc_layer12(h4_2.reshape(4 * NPAD, 128), alT2, srcs, dsts, rs)

    ws3, wd3 = _aug_w(W3, a_src3, a_dst3, OUT)
    W3p = jnp.concatenate(
        [W3.reshape(4, 128, 40), ws3.reshape(4, 128, 8),
         wd3.reshape(4, 128, 8), jnp.zeros((4, 128, 72), F32)], axis=2)
    h3w, al3 = _tc_mm3(out2.reshape(4, NPAD, 128), W3p, b2.reshape(4, 128))
    alT3 = _tc_transpose(al3).reshape(-1)
    out3 = _sc_layer3(h3w, alT3, srcs, dsts, rs)

    b3p = jnp.tile(jnp.pad(b3, (0, 3)).reshape(1, 8), (8, 1))
    res = _tc_final(out3.reshape(2, NPAD, 64), b3p)
    return res[:N, :OUT]


# speedup vs baseline: 7.4478x; 1.0945x over previous
import functools

import jax
import jax.numpy as jnp
from jax import lax
from jax.experimental import pallas as pl
from jax.experimental.pallas import tpu as pltpu
from jax.experimental.pallas import tpu_sc as plsc

N = 10000
E = 320000
IN = 12
H = 8
C = 64
OUT = 5

NPAD = 10240
E2 = E + N
NS = 16
E2P = 330752
E2PX = E2P + 256
BN = 512
ROWS_T = NPAD // NS
F32 = jnp.float32
I32 = jnp.int32


def _mm12_body(kb, apply_relu, x_ref, w_ref, b_ref, h4_ref, al_ref):
    acc = jnp.zeros((BN, 640), F32)
    for k in range(kb):
        xk = x_ref[k]
        if apply_relu:
            xk = jnp.maximum(xk + b_ref[k][None, :], 0.0)
        acc = acc + jnp.dot(xk, w_ref[k], preferred_element_type=F32)
    for j in range(4):
        h4_ref[j] = acc[:, 128 * j:128 * (j + 1)]
    al_ref[...] = acc[:, 512:640]


def _tc_mm12(x4, wk, bk, apply_relu):
    kb = x4.shape[0]
    return pl.pallas_call(
        functools.partial(_mm12_body, kb, apply_relu),
        grid=(NPAD // BN,),
        in_specs=[pl.BlockSpec((kb, BN, 128), lambda i: (0, i, 0)),
                  pl.BlockSpec((kb, 128, 640), lambda i: (0, 0, 0)),
                  pl.BlockSpec((kb, 128), lambda i: (0, 0))],
        out_specs=[pl.BlockSpec((4, BN, 128), lambda i: (0, i, 0)),
                   pl.BlockSpec((BN, 128), lambda i: (i, 0))],
        out_shape=[jax.ShapeDtypeStruct((4, NPAD, 128), F32),
                   jax.ShapeDtypeStruct((NPAD, 128), F32)],
    )(x4, wk, bk)


def _mm3_body(x_ref, w_ref, b_ref, h3_ref, al_ref):
    acc = jnp.zeros((BN, 128), F32)
    for k in range(4):
        xk = jnp.maximum(x_ref[k] + b_ref[k][None, :], 0.0)
        acc = acc + jnp.dot(xk, w_ref[k], preferred_element_type=F32)
    z11 = jnp.zeros((BN, 11), F32)
    parts = []
    for h in range(8):
        parts.append(acc[:, 5 * h:5 * h + 5])
        parts.append(z11)
    h3_ref[...] = jnp.concatenate(parts, axis=1)
    al_ref[...] = acc


def _tc_mm3(x4, wk, bk):
    return pl.pallas_call(
        _mm3_body,
        grid=(NPAD // BN,),
        in_specs=[pl.BlockSpec((4, BN, 128), lambda i: (0, i, 0)),
                  pl.BlockSpec((4, 128, 128), lambda i: (0, 0, 0)),
                  pl.BlockSpec((4, 128), lambda i: (0, 0))],
        out_specs=[pl.BlockSpec((BN, 128), lambda i: (i, 0)),
                   pl.BlockSpec((BN, 128), lambda i: (i, 0))],
        out_shape=[jax.ShapeDtypeStruct((NPAD, 128), F32),
                   jax.ShapeDtypeStruct((NPAD, 128), F32)],
    )(x4, wk, bk)


def _tr_body(x_ref, o_ref):
    o_ref[...] = x_ref[...].T


def _tc_transpose(al):
    return pl.pallas_call(
        _tr_body,
        grid=(NPAD // BN,),
        in_specs=[pl.BlockSpec((BN, 128), lambda i: (i, 0))],
        out_specs=pl.BlockSpec((128, BN), lambda i: (0, i)),
        out_shape=jax.ShapeDtypeStruct((128, NPAD), F32),
    )(al)


def _fin_body(x_ref, b_ref, o_ref):
    sc2 = x_ref[0] + x_ref[1]
    hsum = (sc2[:, 0:8] + sc2[:, 16:24] + sc2[:, 32:40] + sc2[:, 48:56])
    z = hsum * 0.125 + b_ref[0:1, :]
    msk = lax.broadcasted_iota(I32, (1, 8), 1) < OUT
    zm = jnp.where(msk, z, -1e30)
    m = jnp.max(zm, axis=1, keepdims=True)
    lse = jnp.log(jnp.sum(jnp.where(msk, jnp.exp(z - m), 0.0),
                          axis=1, keepdims=True)) + m
    o_ref[...] = z - lse


def _tc_final(x4, b3p):
    return pl.pallas_call(
        _fin_body,
        grid=(NPAD // BN,),
        in_specs=[pl.BlockSpec((2, BN, 64), lambda i: (0, i, 0)),
                  pl.BlockSpec((8, 8), lambda i: (0, 0))],
        out_specs=pl.BlockSpec((BN, 8), lambda i: (i, 0)),
        out_shape=jax.ShapeDtypeStruct((NPAD, 8), F32),
    )(x4, b3p)


CH = 48


def _sc_edge_kernel(a_s_off, a_d_off):
    D = 128
    mesh = plsc.VectorSubcoreMesh(core_axis_name="c", subcore_axis_name="s")

    @functools.partial(
        pl.kernel,
        mesh=mesh,
        compiler_params=pltpu.CompilerParams(needs_layout_passes=False),
        out_type=jax.ShapeDtypeStruct((4 * NPAD, D), F32),
        scratch_types=[
            pltpu.VMEM((16,), I32),
            pltpu.VMEM((16,), I32),
            [pltpu.VMEM((CH,), I32)] * 2,
            [pltpu.VMEM((CH,), I32)] * 2,
            [pltpu.VMEM((CH,), I32)] * 2,
            [pltpu.VMEM((CH, D), F32)] * 2,
            [pltpu.VMEM((2 * CH,), F32)] * 2,
            pltpu.VMEM((ROWS_T * 16,), F32),
            pltpu.VMEM((NPAD,), F32),
            pltpu.VMEM((NPAD,), F32),
            pltpu.VMEM((ROWS_T,), F32),
            pltpu.VMEM((ROWS_T,), F32),
            pltpu.VMEM((ROWS_T, D), F32),
            [pltpu.SemaphoreType.DMA] * 2,
            [pltpu.SemaphoreType.DMA] * 2,
        ],
    )
    def k(h4, alT, srcs, dsts, rs, out4,
          rsb0, rsb1, srcb, dstb, srcb2, rowb, abuf,
          s16, aS0, aS1, aDl0, aDl1, acc, smA, smR):
        cid = lax.axis_index("c")
        tid = lax.axis_index("s")
        base = tid * ROWS_T
        iot = lax.iota(I32, 16)
        zv = jnp.zeros((16,), F32)

        pltpu.sync_copy(rs.at[pl.ds(base, 16)], rsb0)
        pltpu.sync_copy(rs.at[pl.ds(base + ROWS_T, 16)], rsb1)
        e0 = rsb0[...][0]
        e1 = rsb1[...][0]
        a0 = (e0 // 8) * 8
        nch = (e1 - a0 + CH - 1) // CH
        nchp = (nch + 1) // 2

        def issue_sd(ci, b):
            off = a0 + ci * CH
            pltpu.async_copy(srcs.at[pl.ds(off, CH)], srcb[b], smA[b])
            pltpu.async_copy(dsts.at[pl.ds(off, CH)], dstb[b], smA[b])

        def wait_sd(b):
            pltpu.make_async_copy(srcs.at[pl.ds(0, CH)], srcb[b], smA[b]).wait()
            pltpu.make_async_copy(dsts.at[pl.ds(0, CH)], dstb[b], smA[b]).wait()

        def zero_s(i, _):
            s16[pl.ds(i * 16, 16)] = zv
            return 0
        lax.fori_loop(0, ROWS_T, zero_s, 0)

        def round_body(r, _):
            q = cid * 2 + r

            pltpu.sync_copy(alT.at[pl.ds((a_s_off + 2 * q) * NPAD, NPAD)], aS0)
            pltpu.sync_copy(alT.at[pl.ds((a_s_off + 2 * q + 1) * NPAD, NPAD)], aS1)
            pltpu.sync_copy(
                alT.at[pl.ds((a_d_off + 2 * q) * NPAD + base, ROWS_T)], aDl0)
            pltpu.sync_copy(
                alT.at[pl.ds((a_d_off + 2 * q + 1) * NPAD + base, ROWS_T)], aDl1)
            oh0 = jnp.where(iot == 2 * r, 1.0, 0.0)
            oh1 = jnp.where(iot == 2 * r + 1, 1.0, 0.0)

            issue_sd(0, 0)

            def pair_a(cj, _):
                for b in range(2):
                    ci = cj * 2 + b
                    wait_sd(b)
                    issue_sd(ci + 1, 1 - b)
                    off = a0 + ci * CH
                    for g in range(3):
                        ge = off + g * 16 + iot
                        ownf = jnp.where(
                            jnp.logical_and(ge >= e0, ge < e1), 1.0, 0.0)
                        sv = srcb[b][pl.ds(g * 16, 16)]
                        dv = dstb[b][pl.ds(g * 16, 16)]
                        dlv = jnp.clip(dv - base, 0, ROWS_T - 1)
                        e_0 = (plsc.load_gather(aS0, [sv])
                               + plsc.load_gather(aDl0, [dlv]))
                        e_0 = jnp.where(e_0 > 0, e_0, 0.2 * e_0)
                        pm0 = jnp.exp(e_0) * ownf
                        e_1 = (plsc.load_gather(aS1, [sv])
                               + plsc.load_gather(aDl1, [dlv]))
                        e_1 = jnp.where(e_1 > 0, e_1, 0.2 * e_1)
                        pm1 = jnp.exp(e_1) * ownf
                        slv = dlv * 16
                        for i in range(16):
                            sl = slv[i]
                            s16[pl.ds(sl, 16)] = (s16[pl.ds(sl, 16)]
                                                  + pm0[i] * oh0
                                                  + pm1[i] * oh1)
                return 0

            lax.fori_loop(0, nchp, pair_a, 0)
            wait_sd(0)

            def zero_acc(i, _):
                for v in range(D // 16):
                    acc[i, pl.ds(v * 16, 16)] = zv
                return 0
            lax.fori_loop(0, ROWS_T, zero_acc, 0)

            issue_sd(0, 0)

            def pair_b(cj, _):
                for b in range(2):
                    ci = cj * 2 + b
                    wait_sd(b)
                    issue_sd(ci + 1, 1 - b)
                    off = a0 + ci * CH
                    for g in range(3):
                        srcb2[b][pl.ds(g * 16, 16)] = (
                            srcb[b][pl.ds(g * 16, 16)] + q * NPAD)
                    pltpu.async_copy(h4.at[srcb2[b]], rowb[b], smR[b])
                    for g in range(3):
                        ge = off + g * 16 + iot
                        ownv = jnp.logical_and(ge >= e0, ge < e1)
                        sv = srcb[b][pl.ds(g * 16, 16)]
                        dv = dstb[b][pl.ds(g * 16, 16)]
                        dlv = jnp.clip(dv - base, 0, ROWS_T - 1)
                        e_0 = (plsc.load_gather(aS0, [sv])
                               + plsc.load_gather(aDl0, [dlv]))
                        e_0 = jnp.where(e_0 > 0, e_0, 0.2 * e_0)
                        p0 = jnp.exp(e_0)
                        e_1 = (plsc.load_gather(aS1, [sv])
                               + plsc.load_gather(aDl1, [dlv]))
                        e_1 = jnp.where(e_1 > 0, e_1, 0.2 * e_1)
                        p1 = jnp.exp(e_1)
                        s0 = plsc.load_gather(s16, [dlv * 16 + 2 * r])
                        s1 = plsc.load_gather(s16, [dlv * 16 + 2 * r + 1])
                        a_0 = jnp.where(ownv, p0 / (s0 + 1e-16), 0.0)
                        a_1 = jnp.where(ownv, p1 / (s1 + 1e-16), 0.0)
                        abuf[b][pl.ds(g * 16, 16)] = a_0
                        abuf[b][pl.ds(CH + g * 16, 16)] = a_1
                    pltpu.make_async_copy(h4.at[srcb2[b]], rowb[b],
                                          smR[b]).wait()
                    for g in range(3):
                        dv = dstb[b][pl.ds(g * 16, 16)]
                        dlv = jnp.clip(dv - base, 0, ROWS_T - 1)
                        for l in range(16):
                            ei = g * 16 + l
                            br0 = plsc.load_gather(
                                abuf[b], [jnp.full((16,), ei, I32)])
                            br1 = plsc.load_gather(
                                abuf[b], [jnp.full((16,), CH + ei, I32)])
                            dl = dlv[l]
                            for v in range(8):
                                sc = br0 if v < 4 else br1
                                acc[dl, pl.ds(v * 16, 16)] = (
                                    acc[dl, pl.ds(v * 16, 16)]
                                    + rowb[b][ei, pl.ds(v * 16, 16)] * sc)
                return 0

            lax.fori_loop(0, nchp, pair_b, 0)
            wait_sd(0)
            pltpu.sync_copy(acc, out4.at[pl.ds(q * NPAD + base, ROWS_T)])
            return 0

        lax.fori_loop(0, 2, round_body, 0)
    return k


_sc_layer12 = _sc_edge_kernel(0, 8)


def _sc_edge3_kernel():
    mesh = plsc.VectorSubcoreMesh(core_axis_name="c", subcore_axis_name="s")

    @functools.partial(
        pl.kernel,
        mesh=mesh,
        compiler_params=pltpu.CompilerParams(needs_layout_passes=False),
        out_type=jax.ShapeDtypeStruct((2 * NPAD * 64,), F32),
        scratch_types=[
            pltpu.VMEM((16,), I32),
            pltpu.VMEM((16,), I32),
            [pltpu.VMEM((CH,), I32)] * 2,
            [pltpu.VMEM((CH,), I32)] * 2,
            [pltpu.VMEM((CH, 128), F32)] * 2,
            [pltpu.VMEM((4 * CH,), F32)] * 2,
            pltpu.VMEM((ROWS_T * 16,), F32),
            [pltpu.VMEM((NPAD,), F32)] * 4,
            [pltpu.VMEM((ROWS_T,), F32)] * 4,
            pltpu.VMEM((ROWS_T * 64,), F32),
            [pltpu.SemaphoreType.DMA] * 2,
            [pltpu.SemaphoreType.DMA] * 2,
        ],
    )
    def k(h3w, alT, srcs, dsts, rs, out3,
          rsb0, rsb1, srcb, dstb, rowb, abuf, s16, aS, aDl, acc, smA, smR):
        cid = lax.axis_index("c")
        tid = lax.axis_index("s")
        base = tid * ROWS_T
        iot = lax.iota(I32, 16)
        zv = jnp.zeros((16,), F32)

        pltpu.sync_copy(rs.at[pl.ds(base, 16)], rsb0)
        pltpu.sync_copy(rs.at[pl.ds(base + ROWS_T, 16)], rsb1)
        e0 = rsb0[...][0]
        e1 = rsb1[...][0]
        a0 = (e0 // 8) * 8
        nch = (e1 - a0 + CH - 1) // CH
        nchp = (nch + 1) // 2

        def issue_sd(ci, b):
            off = a0 + ci * CH
            pltpu.async_copy(srcs.at[pl.ds(off, CH)], srcb[b], smA[b])
            pltpu.async_copy(dsts.at[pl.ds(off, CH)], dstb[b], smA[b])

        def wait_sd(b):
            pltpu.make_async_copy(srcs.at[pl.ds(0, CH)], srcb[b], smA[b]).wait()
            pltpu.make_async_copy(dsts.at[pl.ds(0, CH)], dstb[b], smA[b]).wait()

        def zero_s(i, _):
            s16[pl.ds(i * 16, 16)] = zv
            return 0
        lax.fori_loop(0, ROWS_T, zero_s, 0)

        for lh in range(4):
            pltpu.sync_copy(
                alT.at[pl.ds((40 + 4 * cid + lh) * NPAD, NPAD)], aS[lh])
            pltpu.sync_copy(
                alT.at[pl.ds((48 + 4 * cid + lh) * NPAD + base, ROWS_T)],
                aDl[lh])

        def edge_p(sv, dlv, lh):
            ee = (plsc.load_gather(aS[lh], [sv])
                  + plsc.load_gather(aDl[lh], [dlv]))
            ee = jnp.where(ee > 0, ee, 0.2 * ee)
            return jnp.exp(ee)

        ohs = [jnp.where(iot == lh, 1.0, 0.0) for lh in range(4)]

        issue_sd(0, 0)

        def pair_a(cj, _):
            for b in range(2):
                ci = cj * 2 + b
                wait_sd(b)
                issue_sd(ci + 1, 1 - b)
                off = a0 + ci * CH
                for g in range(3):
                    ge = off + g * 16 + iot
                    ownf = jnp.where(
                        jnp.logical_and(ge >= e0, ge < e1), 1.0, 0.0)
                    sv = srcb[b][pl.ds(g * 16, 16)]
                    dv = dstb[b][pl.ds(g * 16, 16)]
                    dlv = jnp.clip(dv - base, 0, ROWS_T - 1)
                    pm = [edge_p(sv, dlv, lh) * ownf for lh in range(4)]
                    slv = dlv * 16
                    for i in range(16):
                        sl = slv[i]
                        s16[pl.ds(sl, 16)] = (
                            s16[pl.ds(sl, 16)] + pm[0][i] * ohs[0]
                            + pm[1][i] * ohs[1] + pm[2][i] * ohs[2]
                            + pm[3][i] * ohs[3])
            return 0

        lax.fori_loop(0, nchp, pair_a, 0)
        wait_sd(0)

        def zero_acc(i, _):
            for v in range(4):
                acc[pl.ds(i * 64 + v * 16, 16)] = zv
            return 0
        lax.fori_loop(0, ROWS_T, zero_acc, 0)

        issue_sd(0, 0)

        def pair_b(cj, _):
            for b in range(2):
                ci = cj * 2 + b
                wait_sd(b)
                issue_sd(ci + 1, 1 - b)
                off = a0 + ci * CH
                pltpu.async_copy(h3w.at[srcb[b]], rowb[b], smR[b])
                for g in range(3):
                    ge = off + g * 16 + iot
                    ownv = jnp.logical_and(ge >= e0, ge < e1)
                    sv = srcb[b][pl.ds(g * 16, 16)]
                    dv = dstb[b][pl.ds(g * 16, 16)]
                    dlv = jnp.clip(dv - base, 0, ROWS_T - 1)
                    for lh in range(4):
                        pv = edge_p(sv, dlv, lh)
                        sg = plsc.load_gather(s16, [dlv * 16 + lh])
                        av = jnp.where(ownv, pv / (sg + 1e-16), 0.0)
                        abuf[b][pl.ds(lh * CH + g * 16, 16)] = av
                pltpu.make_async_copy(h3w.at[srcb[b]], rowb[b], smR[b]).wait()
                for g in range(3):
                    dv = dstb[b][pl.ds(g * 16, 16)]
                    dlv = jnp.clip(dv - base, 0, ROWS_T - 1)
                    for l in range(16):
                        ei = g * 16 + l
                        dl = dlv[l]
                        for lh in range(4):
                            br = plsc.load_gather(
                                abuf[b], [jnp.full((16,), lh * CH + ei, I32)])
                            col = 64 * cid + 16 * lh
                            acc[pl.ds(dl * 64 + 16 * lh, 16)] = (
                                acc[pl.ds(dl * 64 + 16 * lh, 16)]
                                + rowb[b][ei, pl.ds(col, 16)] * br)
            return 0

        lax.fori_loop(0, nchp, pair_b, 0)
        wait_sd(0)
        pltpu.sync_copy(acc, out3.at[pl.ds((cid * NPAD + base) * 64,
                                           ROWS_T * 64)])

    return k


_sc_layer3 = _sc_edge3_kernel()


def _aug_w(W, a_s, a_d, ch):
    K = W.shape[0]
    Wr = W.reshape(K, H, ch)
    ws = jnp.einsum("khc,hc->kh", Wr, a_s)
    wd = jnp.einsum("khc,hc->kh", Wr, a_d)
    return ws, wd


def kernel(x, edge_index, W1, a_src1, a_dst1, b1,
           W2, a_src2, a_dst2, b2, W3, a_src3, a_dst3, b3):
    loops = jnp.arange(N, dtype=jnp.int32)
    padi = jnp.full((E2P - E2,), N, jnp.int32)
    src0 = jnp.concatenate([edge_index[0].astype(jnp.int32), loops, padi])
    dst0 = jnp.concatenate([edge_index[1].astype(jnp.int32), loops, padi])
    order = jnp.argsort(dst0)
    dsts = jnp.concatenate([dst0[order],
                            jnp.full((256,), NPAD - 1, jnp.int32)])
    srcs = jnp.concatenate([src0[order], jnp.full((256,), N, jnp.int32)])
    rs = jnp.searchsorted(dsts[:E2P], jnp.arange(NPAD + 1, dtype=jnp.int32)
                          ).astype(jnp.int32)
    rs = jnp.concatenate([rs, jnp.full((15,), E2P, jnp.int32)])

    xp = jnp.zeros((1, NPAD, 128), F32).at[0, :N, :IN].set(x)
    ws1, wd1 = _aug_w(W1, a_src1, a_dst1, C)
    W1p = jnp.zeros((1, 128, 640), F32)
    W1p = W1p.at[0, :IN, :512].set(W1).at[0, :IN, 512:520].set(ws1)
    W1p = W1p.at[0, :IN, 520:528].set(wd1)
    b0 = jnp.zeros((1, 128), F32)
    h4_1, al1 = _tc_mm12(xp, W1p, b0, False)
    alT1 = _tc_transpose(al1).reshape(-1)
    out1 = _sc_layer12(h4_1.reshape(4 * NPAD, 128), alT1, srcs, dsts, rs)

    ws2, wd2 = _aug_w(W2, a_src2, a_dst2, C)
    W2p = jnp.concatenate(
        [W2.reshape(4, 128, 512), ws2.reshape(4, 128, 8),
         wd2.reshape(4, 128, 8), jnp.zeros((4, 128, 112), F32)], axis=2)
    h4_2, al2 = _tc_mm12(out1.reshape(4, NPAD, 128), W2p,
                         b1.reshape(4, 128), True)
    alT2 = _tc_transpose(al2).reshape(-1)
    out2 = _sc_layer12(h4_2.reshape(4 * NPAD, 128), alT2, srcs, dsts, rs)

    ws3, wd3 = _aug_w(W3, a_src3, a_dst3, OUT)
    W3p = jnp.concatenate(
        [W3.reshape(4, 128, 40), ws3.reshape(4, 128, 8),
         wd3.reshape(4, 128, 8), jnp.zeros((4, 128, 72), F32)], axis=2)
    h3w, al3 = _tc_mm3(out2.reshape(4, NPAD, 128), W3p, b2.reshape(4, 128))
    alT3 = _tc_transpose(al3).reshape(-1)
    out3 = _sc_layer3(h3w, alT3, srcs, dsts, rs)

    b3p = jnp.tile(jnp.pad(b3, (0, 3)).reshape(1, 8), (8, 1))
    res = _tc_final(out3.reshape(2, NPAD, 64), b3p)
    return res[:N, :OUT]


# speedup vs baseline: 7.4709x; 1.0031x over previous
import functools

import jax
import jax.numpy as jnp
from jax import lax
from jax.experimental import pallas as pl
from jax.experimental.pallas import tpu as pltpu
from jax.experimental.pallas import tpu_sc as plsc

N = 10000
E = 320000
IN = 12
H = 8
C = 64
OUT = 5

NPAD = 10240
E2 = E + N
NS = 16
E2P = 330752
E2PX = E2P + 256
BN = 512
ROWS_T = NPAD // NS
F32 = jnp.float32
I32 = jnp.int32


def _mm12_body(kb, apply_relu, x_ref, w_ref, b_ref, h4_ref, al_ref):
    acc = jnp.zeros((BN, 640), F32)
    for k in range(kb):
        xk = x_ref[k]
        if apply_relu:
            xk = jnp.maximum(xk + b_ref[k][None, :], 0.0)
        acc = acc + jnp.dot(xk, w_ref[k], preferred_element_type=F32)
    for j in range(4):
        h4_ref[j] = acc[:, 128 * j:128 * (j + 1)]
    al_ref[...] = acc[:, 512:640]


def _tc_mm12(x4, wk, bk, apply_relu):
    kb = x4.shape[0]
    return pl.pallas_call(
        functools.partial(_mm12_body, kb, apply_relu),
        grid=(NPAD // BN,),
        in_specs=[pl.BlockSpec((kb, BN, 128), lambda i: (0, i, 0)),
                  pl.BlockSpec((kb, 128, 640), lambda i: (0, 0, 0)),
                  pl.BlockSpec((kb, 128), lambda i: (0, 0))],
        out_specs=[pl.BlockSpec((4, BN, 128), lambda i: (0, i, 0)),
                   pl.BlockSpec((BN, 128), lambda i: (i, 0))],
        out_shape=[jax.ShapeDtypeStruct((4, NPAD, 128), F32),
                   jax.ShapeDtypeStruct((NPAD, 128), F32)],
    )(x4, wk, bk)


def _mm3_body(x_ref, w_ref, b_ref, h3_ref, al_ref):
    acc = jnp.zeros((BN, 128), F32)
    for k in range(4):
        xk = jnp.maximum(x_ref[k] + b_ref[k][None, :], 0.0)
        acc = acc + jnp.dot(xk, w_ref[k], preferred_element_type=F32)
    z11 = jnp.zeros((BN, 11), F32)
    parts = []
    for h in range(8):
        parts.append(acc[:, 5 * h:5 * h + 5])
        parts.append(z11)
    h3_ref[...] = jnp.concatenate(parts, axis=1)
    al_ref[...] = acc


def _tc_mm3(x4, wk, bk):
    return pl.pallas_call(
        _mm3_body,
        grid=(NPAD // BN,),
        in_specs=[pl.BlockSpec((4, BN, 128), lambda i: (0, i, 0)),
                  pl.BlockSpec((4, 128, 128), lambda i: (0, 0, 0)),
                  pl.BlockSpec((4, 128), lambda i: (0, 0))],
        out_specs=[pl.BlockSpec((BN, 128), lambda i: (i, 0)),
                   pl.BlockSpec((BN, 128), lambda i: (i, 0))],
        out_shape=[jax.ShapeDtypeStruct((NPAD, 128), F32),
                   jax.ShapeDtypeStruct((NPAD, 128), F32)],
    )(x4, wk, bk)


def _tr_body(x_ref, o_ref):
    o_ref[...] = x_ref[...].T


def _tc_transpose(al):
    return pl.pallas_call(
        _tr_body,
        grid=(NPAD // BN,),
        in_specs=[pl.BlockSpec((BN, 128), lambda i: (i, 0))],
        out_specs=pl.BlockSpec((128, BN), lambda i: (0, i)),
        out_shape=jax.ShapeDtypeStruct((128, NPAD), F32),
    )(al)


def _fin_body(x_ref, b_ref, o_ref):
    sc2 = x_ref[0] + x_ref[1]
    hsum = (sc2[:, 0:8] + sc2[:, 16:24] + sc2[:, 32:40] + sc2[:, 48:56])
    z = hsum * 0.125 + b_ref[0:1, :]
    msk = lax.broadcasted_iota(I32, (1, 8), 1) < OUT
    zm = jnp.where(msk, z, -1e30)
    m = jnp.max(zm, axis=1, keepdims=True)
    lse = jnp.log(jnp.sum(jnp.where(msk, jnp.exp(z - m), 0.0),
                          axis=1, keepdims=True)) + m
    o_ref[...] = z - lse


def _tc_final(x4, b3p):
    return pl.pallas_call(
        _fin_body,
        grid=(NPAD // BN,),
        in_specs=[pl.BlockSpec((2, BN, 64), lambda i: (0, i, 0)),
                  pl.BlockSpec((8, 8), lambda i: (0, 0))],
        out_specs=pl.BlockSpec((BN, 8), lambda i: (i, 0)),
        out_shape=jax.ShapeDtypeStruct((NPAD, 8), F32),
    )(x4, b3p)


CH = 64


def _sc_edge_kernel(a_s_off, a_d_off):
    D = 128
    mesh = plsc.VectorSubcoreMesh(core_axis_name="c", subcore_axis_name="s")

    @functools.partial(
        pl.kernel,
        mesh=mesh,
        compiler_params=pltpu.CompilerParams(needs_layout_passes=False),
        out_type=jax.ShapeDtypeStruct((4 * NPAD, D), F32),
        scratch_types=[
            pltpu.VMEM((16,), I32),
            pltpu.VMEM((16,), I32),
            [pltpu.VMEM((CH,), I32)] * 2,
            [pltpu.VMEM((CH,), I32)] * 2,
            [pltpu.VMEM((CH,), I32)] * 2,
            pltpu.VMEM((CH, D), F32),
            [pltpu.VMEM((2 * CH,), F32)] * 2,
            pltpu.VMEM((ROWS_T * 16,), F32),
            pltpu.VMEM((NPAD,), F32),
            pltpu.VMEM((NPAD,), F32),
            pltpu.VMEM((ROWS_T,), F32),
            pltpu.VMEM((ROWS_T,), F32),
            pltpu.VMEM((ROWS_T, D), F32),
            [pltpu.SemaphoreType.DMA] * 2,
            [pltpu.SemaphoreType.DMA] * 2,
        ],
    )
    def k(h4, alT, srcs, dsts, rs, out4,
          rsb0, rsb1, srcb, dstb, srcb2, rowb, abuf,
          s16, aS0, aS1, aDl0, aDl1, acc, smA, smR):
        cid = lax.axis_index("c")
        tid = lax.axis_index("s")
        base = tid * ROWS_T
        iot = lax.iota(I32, 16)
        zv = jnp.zeros((16,), F32)

        pltpu.sync_copy(rs.at[pl.ds(base, 16)], rsb0)
        pltpu.sync_copy(rs.at[pl.ds(base + ROWS_T, 16)], rsb1)
        e0 = rsb0[...][0]
        e1 = rsb1[...][0]
        a0 = (e0 // 8) * 8
        nch = (e1 - a0 + CH - 1) // CH
        nchp = (nch + 1) // 2

        def issue_sd(ci, b):
            off = a0 + ci * CH
            pltpu.async_copy(srcs.at[pl.ds(off, CH)], srcb[b], smA[b])
            pltpu.async_copy(dsts.at[pl.ds(off, CH)], dstb[b], smA[b])

        def wait_sd(b):
            pltpu.make_async_copy(srcs.at[pl.ds(0, CH)], srcb[b], smA[b]).wait()
            pltpu.make_async_copy(dsts.at[pl.ds(0, CH)], dstb[b], smA[b]).wait()

        def zero_s(i, _):
            s16[pl.ds(i * 16, 16)] = zv
            return 0
        lax.fori_loop(0, ROWS_T, zero_s, 0)

        def round_body(r, _):
            q = cid * 2 + r

            pltpu.sync_copy(alT.at[pl.ds((a_s_off + 2 * q) * NPAD, NPAD)], aS0)
            pltpu.sync_copy(alT.at[pl.ds((a_s_off + 2 * q + 1) * NPAD, NPAD)], aS1)
            pltpu.sync_copy(
                alT.at[pl.ds((a_d_off + 2 * q) * NPAD + base, ROWS_T)], aDl0)
            pltpu.sync_copy(
                alT.at[pl.ds((a_d_off + 2 * q + 1) * NPAD + base, ROWS_T)], aDl1)
            oh0 = jnp.where(iot == 2 * r, 1.0, 0.0)
            oh1 = jnp.where(iot == 2 * r + 1, 1.0, 0.0)

            issue_sd(0, 0)

            def pair_a(cj, _):
                for b in range(2):
                    ci = cj * 2 + b
                    wait_sd(b)
                    issue_sd(ci + 1, 1 - b)
                    off = a0 + ci * CH
                    for g in range(CH // 16):
                        ge = off + g * 16 + iot
                        ownf = jnp.where(
                            jnp.logical_and(ge >= e0, ge < e1), 1.0, 0.0)
                        sv = srcb[b][pl.ds(g * 16, 16)]
                        dv = dstb[b][pl.ds(g * 16, 16)]
                        dlv = jnp.clip(dv - base, 0, ROWS_T - 1)
                        e_0 = (plsc.load_gather(aS0, [sv])
                               + plsc.load_gather(aDl0, [dlv]))
                        e_0 = jnp.where(e_0 > 0, e_0, 0.2 * e_0)
                        pm0 = jnp.exp(e_0) * ownf
                        e_1 = (plsc.load_gather(aS1, [sv])
                               + plsc.load_gather(aDl1, [dlv]))
                        e_1 = jnp.where(e_1 > 0, e_1, 0.2 * e_1)
                        pm1 = jnp.exp(e_1) * ownf
                        slv = dlv * 16
                        for i in range(16):
                            sl = slv[i]
                            s16[pl.ds(sl, 16)] = (s16[pl.ds(sl, 16)]
                                                  + pm0[i] * oh0
                                                  + pm1[i] * oh1)
                return 0

            lax.fori_loop(0, nchp, pair_a, 0)
            wait_sd(0)

            def zero_acc(i, _):
                for v in range(D // 16):
                    acc[i, pl.ds(v * 16, 16)] = zv
                return 0
            lax.fori_loop(0, ROWS_T, zero_acc, 0)

            issue_sd(0, 0)

            def pair_b(cj, _):
                for b in range(2):
                    ci = cj * 2 + b
                    wait_sd(b)
                    issue_sd(ci + 1, 1 - b)
                    off = a0 + ci * CH
                    for g in range(CH // 16):
                        srcb2[b][pl.ds(g * 16, 16)] = (
                            srcb[b][pl.ds(g * 16, 16)] + q * NPAD)
                    pltpu.async_copy(h4.at[srcb2[b]], rowb, smR[b])
                    for g in range(CH // 16):
                        ge = off + g * 16 + iot
                        ownv = jnp.logical_and(ge >= e0, ge < e1)
                        sv = srcb[b][pl.ds(g * 16, 16)]
                        dv = dstb[b][pl.ds(g * 16, 16)]
                        dlv = jnp.clip(dv - base, 0, ROWS_T - 1)
                        e_0 = (plsc.load_gather(aS0, [sv])
                               + plsc.load_gather(aDl0, [dlv]))
                        e_0 = jnp.where(e_0 > 0, e_0, 0.2 * e_0)
                        p0 = jnp.exp(e_0)
                        e_1 = (plsc.load_gather(aS1, [sv])
                               + plsc.load_gather(aDl1, [dlv]))
                        e_1 = jnp.where(e_1 > 0, e_1, 0.2 * e_1)
                        p1 = jnp.exp(e_1)
                        s0 = plsc.load_gather(s16, [dlv * 16 + 2 * r])
                        s1 = plsc.load_gather(s16, [dlv * 16 + 2 * r + 1])
                        a_0 = jnp.where(ownv, p0 / (s0 + 1e-16), 0.0)
                        a_1 = jnp.where(ownv, p1 / (s1 + 1e-16), 0.0)
                        abuf[b][pl.ds(g * 16, 16)] = a_0
                        abuf[b][pl.ds(CH + g * 16, 16)] = a_1
                    pltpu.make_async_copy(h4.at[srcb2[b]], rowb,
                                          smR[b]).wait()
                    for g in range(CH // 16):
                        dv = dstb[b][pl.ds(g * 16, 16)]
                        dlv = jnp.clip(dv - base, 0, ROWS_T - 1)
                        for l in range(16):
                            ei = g * 16 + l
                            br0 = plsc.load_gather(
                                abuf[b], [jnp.full((16,), ei, I32)])
                            br1 = plsc.load_gather(
                                abuf[b], [jnp.full((16,), CH + ei, I32)])
                            dl = dlv[l]
                            for v in range(8):
                                sc = br0 if v < 4 else br1
                                acc[dl, pl.ds(v * 16, 16)] = (
                                    acc[dl, pl.ds(v * 16, 16)]
                                    + rowb[ei, pl.ds(v * 16, 16)] * sc)
                return 0

            lax.fori_loop(0, nchp, pair_b, 0)
            wait_sd(0)
            pltpu.sync_copy(acc, out4.at[pl.ds(q * NPAD + base, ROWS_T)])
            return 0

        lax.fori_loop(0, 2, round_body, 0)
    return k


_sc_layer12 = _sc_edge_kernel(0, 8)


def _sc_edge3_kernel():
    mesh = plsc.VectorSubcoreMesh(core_axis_name="c", subcore_axis_name="s")

    @functools.partial(
        pl.kernel,
        mesh=mesh,
        compiler_params=pltpu.CompilerParams(needs_layout_passes=False),
        out_type=jax.ShapeDtypeStruct((2 * NPAD * 64,), F32),
        scratch_types=[
            pltpu.VMEM((16,), I32),
            pltpu.VMEM((16,), I32),
            [pltpu.VMEM((CH,), I32)] * 2,
            [pltpu.VMEM((CH,), I32)] * 2,
            [pltpu.VMEM((CH, 128), F32)] * 2,
            [pltpu.VMEM((4 * CH,), F32)] * 2,
            pltpu.VMEM((ROWS_T * 16,), F32),
            [pltpu.VMEM((NPAD,), F32)] * 4,
            [pltpu.VMEM((ROWS_T,), F32)] * 4,
            pltpu.VMEM((ROWS_T * 64,), F32),
            [pltpu.SemaphoreType.DMA] * 2,
            [pltpu.SemaphoreType.DMA] * 2,
        ],
    )
    def k(h3w, alT, srcs, dsts, rs, out3,
          rsb0, rsb1, srcb, dstb, rowb, abuf, s16, aS, aDl, acc, smA, smR):
        cid = lax.axis_index("c")
        tid = lax.axis_index("s")
        base = tid * ROWS_T
        iot = lax.iota(I32, 16)
        zv = jnp.zeros((16,), F32)

        pltpu.sync_copy(rs.at[pl.ds(base, 16)], rsb0)
        pltpu.sync_copy(rs.at[pl.ds(base + ROWS_T, 16)], rsb1)
        e0 = rsb0[...][0]
        e1 = rsb1[...][0]
        a0 = (e0 // 8) * 8
        nch = (e1 - a0 + CH - 1) // CH
        nchp = (nch + 1) // 2

        def issue_sd(ci, b):
            off = a0 + ci * CH
            pltpu.async_copy(srcs.at[pl.ds(off, CH)], srcb[b], smA[b])
            pltpu.async_copy(dsts.at[pl.ds(off, CH)], dstb[b], smA[b])

        def wait_sd(b):
            pltpu.make_async_copy(srcs.at[pl.ds(0, CH)], srcb[b], smA[b]).wait()
            pltpu.make_async_copy(dsts.at[pl.ds(0, CH)], dstb[b], smA[b]).wait()

        def zero_s(i, _):
            s16[pl.ds(i * 16, 16)] = zv
            return 0
        lax.fori_loop(0, ROWS_T, zero_s, 0)

        for lh in range(4):
            pltpu.sync_copy(
                alT.at[pl.ds((40 + 4 * cid + lh) * NPAD, NPAD)], aS[lh])
            pltpu.sync_copy(
                alT.at[pl.ds((48 + 4 * cid + lh) * NPAD + base, ROWS_T)],
                aDl[lh])

        def edge_p(sv, dlv, lh):
            ee = (plsc.load_gather(aS[lh], [sv])
                  + plsc.load_gather(aDl[lh], [dlv]))
            ee = jnp.where(ee > 0, ee, 0.2 * ee)
            return jnp.exp(ee)

        ohs = [jnp.where(iot == lh, 1.0, 0.0) for lh in range(4)]

        issue_sd(0, 0)

        def pair_a(cj, _):
            for b in range(2):
                ci = cj * 2 + b
                wait_sd(b)
                issue_sd(ci + 1, 1 - b)
                off = a0 + ci * CH
                for g in range(CH // 16):
                    ge = off + g * 16 + iot
                    ownf = jnp.where(
                        jnp.logical_and(ge >= e0, ge < e1), 1.0, 0.0)
                    sv = srcb[b][pl.ds(g * 16, 16)]
                    dv = dstb[b][pl.ds(g * 16, 16)]
                    dlv = jnp.clip(dv - base, 0, ROWS_T - 1)
                    pm = [edge_p(sv, dlv, lh) * ownf for lh in range(4)]
                    slv = dlv * 16
                    for i in range(16):
                        sl = slv[i]
                        s16[pl.ds(sl, 16)] = (
                            s16[pl.ds(sl, 16)] + pm[0][i] * ohs[0]
                            + pm[1][i] * ohs[1] + pm[2][i] * ohs[2]
                            + pm[3][i] * ohs[3])
            return 0

        lax.fori_loop(0, nchp, pair_a, 0)
        wait_sd(0)

        def zero_acc(i, _):
            for v in range(4):
                acc[pl.ds(i * 64 + v * 16, 16)] = zv
            return 0
        lax.fori_loop(0, ROWS_T, zero_acc, 0)

        issue_sd(0, 0)

        def pair_b(cj, _):
            for b in range(2):
                ci = cj * 2 + b
                wait_sd(b)
                issue_sd(ci + 1, 1 - b)
                off = a0 + ci * CH
                pltpu.async_copy(h3w.at[srcb[b]], rowb[b], smR[b])
                for g in range(CH // 16):
                    ge = off + g * 16 + iot
                    ownv = jnp.logical_and(ge >= e0, ge < e1)
                    sv = srcb[b][pl.ds(g * 16, 16)]
                    dv = dstb[b][pl.ds(g * 16, 16)]
                    dlv = jnp.clip(dv - base, 0, ROWS_T - 1)
                    for lh in range(4):
                        pv = edge_p(sv, dlv, lh)
                        sg = plsc.load_gather(s16, [dlv * 16 + lh])
                        av = jnp.where(ownv, pv / (sg + 1e-16), 0.0)
                        abuf[b][pl.ds(lh * CH + g * 16, 16)] = av
                pltpu.make_async_copy(h3w.at[srcb[b]], rowb[b], smR[b]).wait()
                for g in range(CH // 16):
                    dv = dstb[b][pl.ds(g * 16, 16)]
                    dlv = jnp.clip(dv - base, 0, ROWS_T - 1)
                    for l in range(16):
                        ei = g * 16 + l
                        dl = dlv[l]
                        for lh in range(4):
                            br = plsc.load_gather(
                                abuf[b], [jnp.full((16,), lh * CH + ei, I32)])
                            col = 64 * cid + 16 * lh
                            acc[pl.ds(dl * 64 + 16 * lh, 16)] = (
                                acc[pl.ds(dl * 64 + 16 * lh, 16)]
                                + rowb[b][ei, pl.ds(col, 16)] * br)
            return 0

        lax.fori_loop(0, nchp, pair_b, 0)
        wait_sd(0)
        pltpu.sync_copy(acc, out3.at[pl.ds((cid * NPAD + base) * 64,
                                           ROWS_T * 64)])

    return k


_sc_layer3 = _sc_edge3_kernel()


def _aug_w(W, a_s, a_d, ch):
    K = W.shape[0]
    Wr = W.reshape(K, H, ch)
    ws = jnp.einsum("khc,hc->kh", Wr, a_s)
    wd = jnp.einsum("khc,hc->kh", Wr, a_d)
    return ws, wd


def kernel(x, edge_index, W1, a_src1, a_dst1, b1,
           W2, a_src2, a_dst2, b2, W3, a_src3, a_dst3, b3):
    loops = jnp.arange(N, dtype=jnp.int32)
    padi = jnp.full((E2P - E2,), N, jnp.int32)
    src0 = jnp.concatenate([edge_index[0].astype(jnp.int32), loops, padi])
    dst0 = jnp.concatenate([edge_index[1].astype(jnp.int32), loops, padi])
    order = jnp.argsort(dst0)
    dsts = jnp.concatenate([dst0[order],
                            jnp.full((256,), NPAD - 1, jnp.int32)])
    srcs = jnp.concatenate([src0[order], jnp.full((256,), N, jnp.int32)])
    rs = jnp.searchsorted(dsts[:E2P], jnp.arange(NPAD + 1, dtype=jnp.int32)
                          ).astype(jnp.int32)
    rs = jnp.concatenate([rs, jnp.full((15,), E2P, jnp.int32)])

    xp = jnp.zeros((1, NPAD, 128), F32).at[0, :N, :IN].set(x)
    ws1, wd1 = _aug_w(W1, a_src1, a_dst1, C)
    W1p = jnp.zeros((1, 128, 640), F32)
    W1p = W1p.at[0, :IN, :512].set(W1).at[0, :IN, 512:520].set(ws1)
    W1p = W1p.at[0, :IN, 520:528].set(wd1)
    b0 = jnp.zeros((1, 128), F32)
    h4_1, al1 = _tc_mm12(xp, W1p, b0, False)
    alT1 = _tc_transpose(al1).reshape(-1)
    out1 = _sc_layer12(h4_1.reshape(4 * NPAD, 128), alT1, srcs, dsts, rs)

    ws2, wd2 = _aug_w(W2, a_src2, a_dst2, C)
    W2p = jnp.concatenate(
        [W2.reshape(4, 128, 512), ws2.reshape(4, 128, 8),
         wd2.reshape(4, 128, 8), jnp.zeros((4, 128, 112), F32)], axis=2)
    h4_2, al2 = _tc_mm12(out1.reshape(4, NPAD, 128), W2p,
                         b1.reshape(4, 128), True)
    alT2 = _tc_transpose(al2).reshape(-1)
    out2 = _sc_layer12(h4_2.reshape(4 * NPAD, 128), alT2, srcs, dsts, rs)

    ws3, wd3 = _aug_w(W3, a_src3, a_dst3, OUT)
    W3p = jnp.concatenate(
        [W3.reshape(4, 128, 40), ws3.reshape(4, 128, 8),
         wd3.reshape(4, 128, 8), jnp.zeros((4, 128, 72), F32)], axis=2)
    h3w, al3 = _tc_mm3(out2.reshape(4, NPAD, 128), W3p, b2.reshape(4, 128))
    alT3 = _tc_transpose(al3).reshape(-1)
    out3 = _sc_layer3(h3w, alT3, srcs, dsts, rs)

    b3p = jnp.tile(jnp.pad(b3, (0, 3)).reshape(1, 8), (8, 1))
    res = _tc_final(out3.reshape(2, NPAD, 64), b3p)
    return res[:N, :OUT]
